# baseline (speedup 1.0000x reference)
import functools

import numpy as np
import jax
import jax.numpy as jnp
from jax import lax
from jax.experimental import pallas as pl
from jax.experimental.pallas import tpu as pltpu
from jax.experimental.pallas import tpu_sc as plsc

KH, KW = 3, 3
N_TAP = KH * KW

_NC, _NS = 2, 16
_NW = _NC * _NS


def _grid_offset_np(h, w):
    init = np.stack(np.meshgrid(np.arange(KH), np.arange(KW), indexing="ij"))
    init = init.reshape(-1, 2)[None, None, :, :]
    init = np.tile(init, (h, w, 1, 1)).astype(np.float32)
    off0 = int((KH - 1) / 2.0)
    off1 = int((KW - 1) / 2.0)
    grid = np.meshgrid(np.arange(-off0, h - off0), np.arange(-off1, w - off1),
                       indexing="ij")
    grid = np.stack(grid, axis=-1).astype(np.float32)[:, :, None, :]
    grid = np.tile(grid, (1, 1, N_TAP, 1))
    return grid + init


def _prep_kernel(h, w, m, cw2,
                 img, oy, ox, gy, gx, boff,
                 table_out, idx_out, w00, w01, w10, w11):
    x = img[...].astype(jnp.bfloat16)
    lo = lax.bitcast_convert_type(x[:, :cw2], jnp.uint16).astype(jnp.uint32)
    hi = lax.bitcast_convert_type(x[:, cw2:], jnp.uint16).astype(jnp.uint32)
    words = lax.bitcast_convert_type((hi << 16) | lo, jnp.int32)
    zrow = jnp.zeros((w + 1, cw2), jnp.int32)
    table_out[:, 0 * cw2:1 * cw2] = words
    table_out[: m - 1, 1 * cw2:2 * cw2] = words[1:]
    table_out[m - 1 :, 1 * cw2:2 * cw2] = zrow[:1]
    table_out[: m - w, 2 * cw2:3 * cw2] = words[w:]
    table_out[m - w :, 2 * cw2:3 * cw2] = zrow[:w]
    table_out[: m - w - 1, 3 * cw2:4 * cw2] = words[w + 1:]
    table_out[m - w - 1 :, 3 * cw2:4 * cw2] = zrow

    cy = jnp.clip(gy[...] + oy[...], 0.0, float(h - 1))
    cx = jnp.clip(gx[...] + ox[...], 0.0, float(w - 1))
    y0f = jnp.floor(cy)
    x0f = jnp.floor(cx)
    fy = cy - y0f
    fx = cx - x0f
    y0 = y0f.astype(jnp.int32)
    x0 = x0f.astype(jnp.int32)
    idx_out[...] = boff[...] + y0 * w + x0
    gy1 = 1.0 - fy
    gx1 = 1.0 - fx
    w00[...] = gy1 * gx1
    w01[...] = gy1 * fx
    w10[...] = fy * gx1
    w11[...] = fy * fx


def _make_gather(m9, cw2, rows_per_worker, chunk):
    nchunk = rows_per_worker // chunk
    nbuf = 4
    mesh = plsc.VectorSubcoreMesh(core_axis_name="c", subcore_axis_name="s")

    @functools.partial(
        pl.kernel,
        out_type=jax.ShapeDtypeStruct((m9, 4 * cw2), jnp.int32),
        mesh=mesh,
        scratch_types=(
            [pltpu.VMEM((chunk,), jnp.int32)] * nbuf
            + [pltpu.VMEM((chunk, 4 * cw2), jnp.int32)] * nbuf
            + [pltpu.SemaphoreType.DMA] * (2 * nbuf)
        ),
    )
    def gather(table_hbm, idx_hbm, out_hbm, *scr):
        idx_refs = scr[:nbuf]
        buf_refs = scr[nbuf:2 * nbuf]
        gsems = scr[2 * nbuf:3 * nbuf]
        wsems = scr[3 * nbuf:4 * nbuf]
        wid = lax.axis_index("s") * _NC + lax.axis_index("c")
        base = wid * rows_per_worker

        gd = [None] * nbuf
        wd = [None] * nbuf

        def start_gather(ch):
            s = ch % nbuf
            pltpu.sync_copy(idx_hbm.at[pl.ds(base + ch * chunk, chunk)],
                            idx_refs[s])
            gd[s] = pltpu.async_copy(table_hbm.at[idx_refs[s]],
                                     buf_refs[s], gsems[s])

        for ch in range(min(nbuf - 1, nchunk)):
            start_gather(ch)
        for ch in range(nchunk):
            s = ch % nbuf
            nxt = ch + nbuf - 1
            if nxt < nchunk:
                sn = nxt % nbuf
                if wd[sn] is not None:
                    wd[sn].wait()
                    wd[sn] = None
                start_gather(nxt)
            gd[s].wait()
            wd[s] = pltpu.async_copy(
                buf_refs[s],
                out_hbm.at[pl.ds(base + ch * chunk, chunk)],
                wsems[s])
        for s in range(nbuf):
            if wd[s] is not None:
                wd[s].wait()

    return gather


def _blend_matmul_kernel(n_tap, cw2, lb, st_ref, wg_ref, wa_ref, wb_ref,
                         b_ref, o_ref):
    acc_a = None
    acc_b = None
    for n in range(n_tap):
        u = pltpu.bitcast(st_ref[n], jnp.bfloat16)
        mapped = (u[:, 0 * cw2:1 * cw2] * wg_ref[n, :, 0:1]
                  + u[:, 1 * cw2:2 * cw2] * wg_ref[n, :, 1:2]
                  + u[:, 2 * cw2:3 * cw2] * wg_ref[n, :, 2:3]
                  + u[:, 3 * cw2:4 * cw2] * wg_ref[n, :, 3:4])
        da = jnp.dot(mapped, wa_ref[n], preferred_element_type=jnp.float32)
        db = jnp.dot(mapped, wb_ref[n], preferred_element_type=jnp.float32)
        acc_a = da if acc_a is None else acc_a + da
        acc_b = db if acc_b is None else acc_b + db
    o_ref[...] = acc_a + pltpu.roll(acc_b, 2 * lb - 1, 0) + b_ref[...]


def kernel(inputs, offsets, W, b):
    bsz, h, w, c_in = inputs.shape
    n_tap, _, c_out = W.shape
    hw = h * w
    m = bsz * hw
    m9 = m * n_tap
    cw2 = c_in // 2

    grid = _grid_offset_np(h, w)
    gy = np.tile(grid[..., 0].transpose(2, 0, 1)[:, None], (1, bsz, 1, 1))
    gx = np.tile(grid[..., 1].transpose(2, 0, 1)[:, None], (1, bsz, 1, 1))
    boff = np.tile(np.repeat(np.arange(bsz, dtype=np.int32) * hw, hw), n_tap)

    lanes = 128
    rows128 = m9 // lanes
    gy = jnp.asarray(gy.reshape(rows128, lanes))
    gx = jnp.asarray(gx.reshape(rows128, lanes))
    boff = jnp.asarray(boff.reshape(rows128, lanes))

    off5 = offsets.reshape(bsz, h, w, n_tap, 2)
    oy = jnp.transpose(off5[..., 0], (3, 0, 1, 2)).reshape(rows128, lanes)
    ox = jnp.transpose(off5[..., 1], (3, 0, 1, 2)).reshape(rows128, lanes)

    shp_f = jax.ShapeDtypeStruct((rows128, lanes), jnp.float32)
    table4, idx2, w00, w01, w10, w11 = pl.pallas_call(
        functools.partial(_prep_kernel, h, w, m, cw2),
        out_shape=(
            jax.ShapeDtypeStruct((m, 4 * cw2), jnp.int32),
            jax.ShapeDtypeStruct((rows128, lanes), jnp.int32),
            shp_f, shp_f, shp_f, shp_f,
        ),
    )(inputs.reshape(m, c_in), oy, ox, gy, gx, boff)

    wgt9 = (jnp.stack([w00, w01, w10, w11], axis=-1)
            .reshape(n_tap, m, 1, 4).astype(jnp.bfloat16))
    wgt9 = jnp.broadcast_to(wgt9, (n_tap, m, 2, 4)).reshape(n_tap, 2 * m, 4)

    wf = W.astype(jnp.bfloat16)
    wa = wf[:, :cw2, :]
    wb = wf[:, cw2:, :]
    b2 = b.reshape(1, c_out)
    lb = 256

    nseg = 4
    m2 = m // nseg
    m9s = m2 * n_tap
    rows_per_worker = m9s // _NW
    chunk = 32
    sc_fn = _make_gather(m9s, cw2, rows_per_worker, chunk)
    idx3 = idx2.reshape(n_tap, m)

    outs = []
    for seg in range(nseg):
        idx_seg = idx3[:, seg * m2:(seg + 1) * m2].reshape(m9s)
        patches = sc_fn(table4, idx_seg).reshape(n_tap, m2, 4 * cw2)
        wg_seg = wgt9[:, 2 * seg * m2:2 * (seg + 1) * m2]
        out_h = pl.pallas_call(
            functools.partial(_blend_matmul_kernel, n_tap, cw2, lb),
            grid=(m2 // lb,),
            in_specs=[
                pl.BlockSpec((n_tap, lb, 4 * cw2), lambda i: (0, i, 0)),
                pl.BlockSpec((n_tap, 2 * lb, 4), lambda i: (0, i, 0)),
                pl.BlockSpec((n_tap, cw2, c_out), lambda i: (0, 0, 0)),
                pl.BlockSpec((n_tap, cw2, c_out), lambda i: (0, 0, 0)),
                pl.BlockSpec((1, c_out), lambda i: (0, 0)),
            ],
            out_specs=pl.BlockSpec((2 * lb, c_out), lambda i: (i, 0)),
            out_shape=jax.ShapeDtypeStruct((2 * m2, c_out), jnp.float32),
        )(patches, wg_seg, wa, wb, b2)
        outs.append(out_h.reshape(m2, 2, c_out)[:, 0, :])
    out = jnp.concatenate(outs)
    return out.reshape(bsz, h, w, c_out)


# speedup vs baseline: 1.0244x; 1.0244x over previous
import functools

import numpy as np
import jax
import jax.numpy as jnp
from jax import lax
from jax.experimental import pallas as pl
from jax.experimental.pallas import tpu as pltpu
from jax.experimental.pallas import tpu_sc as plsc

KH, KW = 3, 3
N_TAP = KH * KW

_NC, _NS = 2, 16
_NW = _NC * _NS


def _grid_offset_np(h, w):
    init = np.stack(np.meshgrid(np.arange(KH), np.arange(KW), indexing="ij"))
    init = init.reshape(-1, 2)[None, None, :, :]
    init = np.tile(init, (h, w, 1, 1)).astype(np.float32)
    off0 = int((KH - 1) / 2.0)
    off1 = int((KW - 1) / 2.0)
    grid = np.meshgrid(np.arange(-off0, h - off0), np.arange(-off1, w - off1),
                       indexing="ij")
    grid = np.stack(grid, axis=-1).astype(np.float32)[:, :, None, :]
    grid = np.tile(grid, (1, 1, N_TAP, 1))
    return grid + init


def _prep_kernel(h, w, m, cw2,
                 img, oy, ox, gy, gx, boff,
                 table_out, idx_out, w00, w01, w10, w11):
    x = img[...].astype(jnp.bfloat16)
    lo = lax.bitcast_convert_type(x[:, :cw2], jnp.uint16).astype(jnp.uint32)
    hi = lax.bitcast_convert_type(x[:, cw2:], jnp.uint16).astype(jnp.uint32)
    words = lax.bitcast_convert_type((hi << 16) | lo, jnp.int32)
    zrow = jnp.zeros((w + 1, cw2), jnp.int32)
    table_out[:, 0 * cw2:1 * cw2] = words
    table_out[: m - 1, 1 * cw2:2 * cw2] = words[1:]
    table_out[m - 1 :, 1 * cw2:2 * cw2] = zrow[:1]
    table_out[: m - w, 2 * cw2:3 * cw2] = words[w:]
    table_out[m - w :, 2 * cw2:3 * cw2] = zrow[:w]
    table_out[: m - w - 1, 3 * cw2:4 * cw2] = words[w + 1:]
    table_out[m - w - 1 :, 3 * cw2:4 * cw2] = zrow

    cy = jnp.clip(gy[...] + oy[...], 0.0, float(h - 1))
    cx = jnp.clip(gx[...] + ox[...], 0.0, float(w - 1))
    y0f = jnp.floor(cy)
    x0f = jnp.floor(cx)
    fy = cy - y0f
    fx = cx - x0f
    y0 = y0f.astype(jnp.int32)
    x0 = x0f.astype(jnp.int32)
    idx_out[...] = boff[...] + y0 * w + x0
    gy1 = 1.0 - fy
    gx1 = 1.0 - fx
    w00[...] = gy1 * gx1
    w01[...] = gy1 * fx
    w10[...] = fy * gx1
    w11[...] = fy * fx


def _make_gather(m9, cw2, rows_per_worker, chunk):
    nchunk = rows_per_worker // chunk
    nbuf = 4
    mesh = plsc.VectorSubcoreMesh(core_axis_name="c", subcore_axis_name="s")

    @functools.partial(
        pl.kernel,
        out_type=jax.ShapeDtypeStruct((m9, 4 * cw2), jnp.int32),
        mesh=mesh,
        scratch_types=(
            [pltpu.VMEM((chunk,), jnp.int32)] * nbuf
            + [pltpu.VMEM((chunk, 4 * cw2), jnp.int32)] * nbuf
            + [pltpu.SemaphoreType.DMA] * (2 * nbuf)
        ),
    )
    def gather(table_hbm, idx_hbm, out_hbm, *scr):
        idx_refs = scr[:nbuf]
        buf_refs = scr[nbuf:2 * nbuf]
        gsems = scr[2 * nbuf:3 * nbuf]
        wsems = scr[3 * nbuf:4 * nbuf]
        wid = lax.axis_index("s") * _NC + lax.axis_index("c")
        base = wid * rows_per_worker

        gd = [None] * nbuf
        wd = [None] * nbuf

        def start_gather(ch):
            s = ch % nbuf
            pltpu.sync_copy(idx_hbm.at[pl.ds(base + ch * chunk, chunk)],
                            idx_refs[s])
            gd[s] = pltpu.async_copy(table_hbm.at[idx_refs[s]],
                                     buf_refs[s], gsems[s])

        for ch in range(min(nbuf - 1, nchunk)):
            start_gather(ch)
        for ch in range(nchunk):
            s = ch % nbuf
            nxt = ch + nbuf - 1
            if nxt < nchunk:
                sn = nxt % nbuf
                if wd[sn] is not None:
                    wd[sn].wait()
                    wd[sn] = None
                start_gather(nxt)
            gd[s].wait()
            wd[s] = pltpu.async_copy(
                buf_refs[s],
                out_hbm.at[pl.ds(base + ch * chunk, chunk)],
                wsems[s])
        for s in range(nbuf):
            if wd[s] is not None:
                wd[s].wait()

    return gather


def _blend_matmul_kernel(n_tap, cw2, lb, st_ref, wg_ref, wa_ref, wb_ref,
                         b_ref, o_ref):
    acc_a = None
    acc_b = None
    for n in range(n_tap):
        u = pltpu.bitcast(st_ref[n], jnp.bfloat16)
        mapped = (u[:, 0 * cw2:1 * cw2] * wg_ref[n, :, 0:1]
                  + u[:, 1 * cw2:2 * cw2] * wg_ref[n, :, 1:2]
                  + u[:, 2 * cw2:3 * cw2] * wg_ref[n, :, 2:3]
                  + u[:, 3 * cw2:4 * cw2] * wg_ref[n, :, 3:4])
        da = jnp.dot(mapped, wa_ref[n], preferred_element_type=jnp.float32)
        db = jnp.dot(mapped, wb_ref[n], preferred_element_type=jnp.float32)
        acc_a = da if acc_a is None else acc_a + da
        acc_b = db if acc_b is None else acc_b + db
    o_ref[...] = acc_a + pltpu.roll(acc_b, 2 * lb - 1, 0) + b_ref[...]


def kernel(inputs, offsets, W, b):
    bsz, h, w, c_in = inputs.shape
    n_tap, _, c_out = W.shape
    hw = h * w
    m = bsz * hw
    m9 = m * n_tap
    cw2 = c_in // 2

    grid = _grid_offset_np(h, w)
    gy = np.tile(grid[..., 0].transpose(2, 0, 1)[:, None], (1, bsz, 1, 1))
    gx = np.tile(grid[..., 1].transpose(2, 0, 1)[:, None], (1, bsz, 1, 1))
    boff = np.tile(np.repeat(np.arange(bsz, dtype=np.int32) * hw, hw), n_tap)

    lanes = 128
    rows128 = m9 // lanes
    gy = jnp.asarray(gy.reshape(rows128, lanes))
    gx = jnp.asarray(gx.reshape(rows128, lanes))
    boff = jnp.asarray(boff.reshape(rows128, lanes))

    off5 = offsets.reshape(bsz, h, w, n_tap, 2)
    oy = jnp.transpose(off5[..., 0], (3, 0, 1, 2)).reshape(rows128, lanes)
    ox = jnp.transpose(off5[..., 1], (3, 0, 1, 2)).reshape(rows128, lanes)

    shp_f = jax.ShapeDtypeStruct((rows128, lanes), jnp.float32)
    table4, idx2, w00, w01, w10, w11 = pl.pallas_call(
        functools.partial(_prep_kernel, h, w, m, cw2),
        out_shape=(
            jax.ShapeDtypeStruct((m, 4 * cw2), jnp.int32),
            jax.ShapeDtypeStruct((rows128, lanes), jnp.int32),
            shp_f, shp_f, shp_f, shp_f,
        ),
    )(inputs.reshape(m, c_in), oy, ox, gy, gx, boff)

    wgt9 = (jnp.stack([w00, w01, w10, w11], axis=-1)
            .reshape(n_tap, m, 1, 4).astype(jnp.bfloat16))
    wgt9 = jnp.broadcast_to(wgt9, (n_tap, m, 2, 4)).reshape(n_tap, 2 * m, 4)

    wf = W.astype(jnp.bfloat16)
    wa = wf[:, :cw2, :]
    wb = wf[:, cw2:, :]
    b2 = b.reshape(1, c_out)
    lb = 512

    nseg = 2
    m2 = m // nseg
    m9s = m2 * n_tap
    rows_per_worker = m9s // _NW
    chunk = 32
    sc_fn = _make_gather(m9s, cw2, rows_per_worker, chunk)
    idx3 = idx2.reshape(n_tap, m)

    outs = []
    for seg in range(nseg):
        idx_seg = idx3[:, seg * m2:(seg + 1) * m2].reshape(m9s)
        patches = sc_fn(table4, idx_seg).reshape(n_tap, m2, 4 * cw2)
        wg_seg = wgt9[:, 2 * seg * m2:2 * (seg + 1) * m2]
        out_h = pl.pallas_call(
            functools.partial(_blend_matmul_kernel, n_tap, cw2, lb),
            grid=(m2 // lb,),
            in_specs=[
                pl.BlockSpec((n_tap, lb, 4 * cw2), lambda i: (0, i, 0)),
                pl.BlockSpec((n_tap, 2 * lb, 4), lambda i: (0, i, 0)),
                pl.BlockSpec((n_tap, cw2, c_out), lambda i: (0, 0, 0)),
                pl.BlockSpec((n_tap, cw2, c_out), lambda i: (0, 0, 0)),
                pl.BlockSpec((1, c_out), lambda i: (0, 0)),
            ],
            out_specs=pl.BlockSpec((2 * lb, c_out), lambda i: (i, 0)),
            out_shape=jax.ShapeDtypeStruct((2 * m2, c_out), jnp.float32),
        )(patches, wg_seg, wa, wb, b2)
        outs.append(out_h.reshape(m2, 2, c_out)[:, 0, :])
    out = jnp.concatenate(outs)
    return out.reshape(bsz, h, w, c_out)


# speedup vs baseline: 1.0399x; 1.0151x over previous
import functools

import numpy as np
import jax
import jax.numpy as jnp
from jax import lax
from jax.experimental import pallas as pl
from jax.experimental.pallas import tpu as pltpu
from jax.experimental.pallas import tpu_sc as plsc

KH, KW = 3, 3
N_TAP = KH * KW

_NC, _NS = 2, 16
_NW = _NC * _NS


def _grid_offset_np(h, w):
    init = np.stack(np.meshgrid(np.arange(KH), np.arange(KW), indexing="ij"))
    init = init.reshape(-1, 2)[None, None, :, :]
    init = np.tile(init, (h, w, 1, 1)).astype(np.float32)
    off0 = int((KH - 1) / 2.0)
    off1 = int((KW - 1) / 2.0)
    grid = np.meshgrid(np.arange(-off0, h - off0), np.arange(-off1, w - off1),
                       indexing="ij")
    grid = np.stack(grid, axis=-1).astype(np.float32)[:, :, None, :]
    grid = np.tile(grid, (1, 1, N_TAP, 1))
    return grid + init


def _prep_kernel(h, w, m, cw2,
                 img, oy, ox, gy, gx, boff,
                 table_out, idx_out, w00, w01, w10, w11):
    x = img[...].astype(jnp.bfloat16)
    lo = lax.bitcast_convert_type(x[:, :cw2], jnp.uint16).astype(jnp.uint32)
    hi = lax.bitcast_convert_type(x[:, cw2:], jnp.uint16).astype(jnp.uint32)
    words = lax.bitcast_convert_type((hi << 16) | lo, jnp.int32)
    zrow = jnp.zeros((w + 1, cw2), jnp.int32)
    table_out[:, 0 * cw2:1 * cw2] = words
    table_out[: m - 1, 1 * cw2:2 * cw2] = words[1:]
    table_out[m - 1 :, 1 * cw2:2 * cw2] = zrow[:1]
    table_out[: m - w, 2 * cw2:3 * cw2] = words[w:]
    table_out[m - w :, 2 * cw2:3 * cw2] = zrow[:w]
    table_out[: m - w - 1, 3 * cw2:4 * cw2] = words[w + 1:]
    table_out[m - w - 1 :, 3 * cw2:4 * cw2] = zrow

    cy = jnp.clip(gy[...] + oy[...], 0.0, float(h - 1))
    cx = jnp.clip(gx[...] + ox[...], 0.0, float(w - 1))
    y0f = jnp.floor(cy)
    x0f = jnp.floor(cx)
    fy = cy - y0f
    fx = cx - x0f
    y0 = y0f.astype(jnp.int32)
    x0 = x0f.astype(jnp.int32)
    idx_out[...] = boff[...] + y0 * w + x0
    gy1 = 1.0 - fy
    gx1 = 1.0 - fx
    w00[...] = gy1 * gx1
    w01[...] = gy1 * fx
    w10[...] = fy * gx1
    w11[...] = fy * fx


def _make_gather(m9, cw2, rows_per_worker, chunk):
    nchunk = rows_per_worker // chunk
    nbuf = 4
    mesh = plsc.VectorSubcoreMesh(core_axis_name="c", subcore_axis_name="s")

    @functools.partial(
        pl.kernel,
        out_type=jax.ShapeDtypeStruct((m9, 4 * cw2), jnp.int32),
        mesh=mesh,
        scratch_types=(
            [pltpu.VMEM((chunk,), jnp.int32)] * nbuf
            + [pltpu.VMEM((chunk, 4 * cw2), jnp.int32)] * nbuf
            + [pltpu.SemaphoreType.DMA] * (2 * nbuf)
        ),
    )
    def gather(table_hbm, idx_hbm, out_hbm, *scr):
        idx_refs = scr[:nbuf]
        buf_refs = scr[nbuf:2 * nbuf]
        gsems = scr[2 * nbuf:3 * nbuf]
        wsems = scr[3 * nbuf:4 * nbuf]
        wid = lax.axis_index("s") * _NC + lax.axis_index("c")
        base = wid * rows_per_worker

        gd = [None] * nbuf
        wd = [None] * nbuf

        def start_gather(ch):
            s = ch % nbuf
            pltpu.sync_copy(idx_hbm.at[pl.ds(base + ch * chunk, chunk)],
                            idx_refs[s])
            gd[s] = pltpu.async_copy(table_hbm.at[idx_refs[s]],
                                     buf_refs[s], gsems[s])

        for ch in range(min(nbuf - 1, nchunk)):
            start_gather(ch)
        for ch in range(nchunk):
            s = ch % nbuf
            nxt = ch + nbuf - 1
            if nxt < nchunk:
                sn = nxt % nbuf
                if wd[sn] is not None:
                    wd[sn].wait()
                    wd[sn] = None
                start_gather(nxt)
            gd[s].wait()
            wd[s] = pltpu.async_copy(
                buf_refs[s],
                out_hbm.at[pl.ds(base + ch * chunk, chunk)],
                wsems[s])
        for s in range(nbuf):
            if wd[s] is not None:
                wd[s].wait()

    return gather


def _blend_matmul_kernel(n_tap, cw2, lb, st_ref, wg_ref, wa_ref, wb_ref,
                         b_ref, o_ref):
    acc_a = None
    acc_b = None
    for n in range(n_tap):
        u = pltpu.bitcast(st_ref[n], jnp.bfloat16)
        mapped = (u[:, 0 * cw2:1 * cw2] * wg_ref[n, :, 0:1]
                  + u[:, 1 * cw2:2 * cw2] * wg_ref[n, :, 1:2]
                  + u[:, 2 * cw2:3 * cw2] * wg_ref[n, :, 2:3]
                  + u[:, 3 * cw2:4 * cw2] * wg_ref[n, :, 3:4])
        da = jnp.dot(mapped, wa_ref[n], preferred_element_type=jnp.float32)
        db = jnp.dot(mapped, wb_ref[n], preferred_element_type=jnp.float32)
        acc_a = da if acc_a is None else acc_a + da
        acc_b = db if acc_b is None else acc_b + db
    o_ref[...] = acc_a + pltpu.roll(acc_b, 2 * lb - 1, 0) + b_ref[...]


def kernel(inputs, offsets, W, b):
    bsz, h, w, c_in = inputs.shape
    n_tap, _, c_out = W.shape
    hw = h * w
    m = bsz * hw
    m9 = m * n_tap
    cw2 = c_in // 2

    grid = _grid_offset_np(h, w)
    gy = np.tile(grid[..., 0].transpose(2, 0, 1)[:, None], (1, bsz, 1, 1))
    gx = np.tile(grid[..., 1].transpose(2, 0, 1)[:, None], (1, bsz, 1, 1))
    boff = np.tile(np.repeat(np.arange(bsz, dtype=np.int32) * hw, hw), n_tap)

    lanes = 128
    rows128 = m9 // lanes
    gy = jnp.asarray(gy.reshape(rows128, lanes))
    gx = jnp.asarray(gx.reshape(rows128, lanes))
    boff = jnp.asarray(boff.reshape(rows128, lanes))

    off5 = offsets.reshape(bsz, h, w, n_tap, 2)
    oy = jnp.transpose(off5[..., 0], (3, 0, 1, 2)).reshape(rows128, lanes)
    ox = jnp.transpose(off5[..., 1], (3, 0, 1, 2)).reshape(rows128, lanes)

    shp_f = jax.ShapeDtypeStruct((rows128, lanes), jnp.float32)
    table4, idx2, w00, w01, w10, w11 = pl.pallas_call(
        functools.partial(_prep_kernel, h, w, m, cw2),
        out_shape=(
            jax.ShapeDtypeStruct((m, 4 * cw2), jnp.int32),
            jax.ShapeDtypeStruct((rows128, lanes), jnp.int32),
            shp_f, shp_f, shp_f, shp_f,
        ),
    )(inputs.reshape(m, c_in), oy, ox, gy, gx, boff)

    wgt9 = (jnp.stack([w00, w01, w10, w11], axis=-1)
            .reshape(n_tap, m, 1, 4).astype(jnp.bfloat16))
    wgt9 = jnp.broadcast_to(wgt9, (n_tap, m, 2, 4)).reshape(n_tap, 2 * m, 4)

    wf = W.astype(jnp.bfloat16)
    wa = wf[:, :cw2, :]
    wb = wf[:, cw2:, :]
    b2 = b.reshape(1, c_out)
    lb = 256

    nseg = 2
    m2 = m // nseg
    m9s = m2 * n_tap
    rows_per_worker = m9s // _NW
    chunk = 32
    sc_fn = _make_gather(m9s, cw2, rows_per_worker, chunk)
    idx3 = idx2.reshape(n_tap, m)

    outs = []
    for seg in range(nseg):
        idx_seg = idx3[:, seg * m2:(seg + 1) * m2].reshape(m9s)
        patches = sc_fn(table4, idx_seg).reshape(n_tap, m2, 4 * cw2)
        wg_seg = wgt9[:, 2 * seg * m2:2 * (seg + 1) * m2]
        out_h = pl.pallas_call(
            functools.partial(_blend_matmul_kernel, n_tap, cw2, lb),
            grid=(m2 // lb,),
            in_specs=[
                pl.BlockSpec((n_tap, lb, 4 * cw2), lambda i: (0, i, 0)),
                pl.BlockSpec((n_tap, 2 * lb, 4), lambda i: (0, i, 0)),
                pl.BlockSpec((n_tap, cw2, c_out), lambda i: (0, 0, 0)),
                pl.BlockSpec((n_tap, cw2, c_out), lambda i: (0, 0, 0)),
                pl.BlockSpec((1, c_out), lambda i: (0, 0)),
            ],
            out_specs=pl.BlockSpec((2 * lb, c_out), lambda i: (i, 0)),
            out_shape=jax.ShapeDtypeStruct((2 * m2, c_out), jnp.float32),
        )(patches, wg_seg, wa, wb, b2)
        outs.append(out_h.reshape(m2, 2, c_out)[:, 0, :])
    out = jnp.concatenate(outs)
    return out.reshape(bsz, h, w, c_out)
